# baseline (speedup 1.0000x reference)
import functools

import jax
import jax.numpy as jnp
from jax import lax
from jax.experimental import pallas as pl
from jax.experimental.pallas import tpu as pltpu
from jax.experimental.pallas import tpu_sc as plsc

N = 16384
K = 10
WIDTH = 64
B = N * K

NC = 2
NS = 16
NW = NC * NS
NSPLIT = 1
BS = B // NSPLIT
B_PER_W = BS // NW
VOCAB = 1001
CH = 320
NCH = B_PER_W // CH
NHBM = 3
NSPM = NCH - NHBM
NBUF = 2


def _gather_kernel(table_hbm, idx_hbm, out_hbm, tab_v, idx_v, bufs, hbufs,
                   gsems, hgsems, wsems, hwsems):
    sid = lax.axis_index("s")
    wid = sid * NC + lax.axis_index("c")
    base = wid * B_PER_W

    @pl.when(sid == 0)
    def _stage_table():
        pltpu.sync_copy(table_hbm, tab_v)

    pltpu.sync_copy(idx_hbm.at[pl.ds(base, B_PER_W)], idx_v)

    def start_gather(src, c, buf, sem):
        return pltpu.async_copy(
            src.at[idx_v.at[pl.ds(c * CH, CH)]], buf, sem
        )

    def start_write(c, buf, sem):
        return pltpu.async_copy(
            buf, out_hbm.at[pl.ds(base + c * CH, CH)], sem
        )

    hghandles = [
        start_gather(table_hbm, NSPM + h, hbufs[h], hgsems[h])
        for h in range(NHBM)
    ]
    plsc.subcore_barrier()

    ghandles = [None] * NBUF
    whandles = [None] * NBUF
    for c in range(NSPM + NBUF - 1):
        if c < NSPM:
            b = c % NBUF
            if whandles[b] is not None:
                whandles[b].wait()
            ghandles[b] = start_gather(tab_v, c, bufs[b], gsems[b])
        d = c - (NBUF - 1)
        if d >= 0:
            db = d % NBUF
            ghandles[db].wait()
            whandles[db] = start_write(d, bufs[db], wsems[db])
    hwhandles = []
    for h in range(NHBM):
        hghandles[h].wait()
        hwhandles.append(start_write(NSPM + h, hbufs[h], hwsems[h]))
    for b in range(NBUF):
        if whandles[b] is not None:
            whandles[b].wait()
    for h in range(NHBM):
        hwhandles[h].wait()


@jax.jit
def _lookup(uuid_values_flat, uuid_embedding):
    mesh = plsc.VectorSubcoreMesh(core_axis_name="c", subcore_axis_name="s")
    k = functools.partial(
        pl.kernel,
        mesh=mesh,
        out_type=jax.ShapeDtypeStruct((BS, WIDTH), jnp.float32),
        scratch_types=[
            pltpu.VMEM_SHARED((VOCAB, WIDTH), jnp.float32),
            pltpu.VMEM((B_PER_W,), jnp.int32),
            [pltpu.VMEM((CH, WIDTH), jnp.float32) for _ in range(NBUF)],
            [pltpu.VMEM((CH, WIDTH), jnp.float32) for _ in range(NHBM)],
            [pltpu.SemaphoreType.DMA for _ in range(NBUF)],
            [pltpu.SemaphoreType.DMA for _ in range(NHBM)],
            [pltpu.SemaphoreType.DMA for _ in range(NBUF)],
            [pltpu.SemaphoreType.DMA for _ in range(NHBM)],
        ],
        compiler_params=pltpu.CompilerParams(use_tc_tiling_on_sc=False),
    )(_gather_kernel)
    parts = [
        k(uuid_embedding, lax.slice(uuid_values_flat, (s * BS,), ((s + 1) * BS,)))
        for s in range(NSPLIT)
    ]
    return jnp.concatenate(parts, axis=0)


def kernel(partname_indices, pos_values, uuid_values, uuid_embedding):
    flat = _lookup(uuid_values.reshape(-1), uuid_embedding)
    return flat.reshape(N, K * WIDTH)


# speedup vs baseline: 1.0497x; 1.0497x over previous
import functools

import jax
import jax.numpy as jnp
from jax import lax
from jax.experimental import pallas as pl
from jax.experimental.pallas import tpu as pltpu
from jax.experimental.pallas import tpu_sc as plsc

N = 16384
K = 10
WIDTH = 64
B = N * K

NC = 2
NS = 16
NW = NC * NS
NSPLIT = 1
BS = B // NSPLIT
B_PER_W = BS // NW
VOCAB = 1001
CH = 320
NCH = B_PER_W // CH
NBUF = 4


def _gather_kernel(table_hbm, idx_hbm, out_hbm, tab_v, idx_v, bufs, gsems,
                   wsems):
    sid = lax.axis_index("s")
    wid = sid * NC + lax.axis_index("c")
    base = wid * B_PER_W

    @pl.when(sid == 0)
    def _stage_table():
        pltpu.sync_copy(table_hbm, tab_v)

    pltpu.sync_copy(idx_hbm.at[pl.ds(base, B_PER_W)], idx_v)
    plsc.subcore_barrier()

    def start_gather(c):
        b = c % NBUF
        return pltpu.async_copy(
            tab_v.at[idx_v.at[pl.ds(c * CH, CH)]], bufs[b], gsems[b]
        )

    def start_write(c):
        b = c % NBUF
        return pltpu.async_copy(
            bufs[b], out_hbm.at[pl.ds(base + c * CH, CH)], wsems[b]
        )

    ghandles = [None] * NBUF
    whandles = [None] * NBUF
    for c in range(NCH + NBUF - 1):
        if c < NCH:
            b = c % NBUF
            if whandles[b] is not None:
                whandles[b].wait()
            ghandles[b] = start_gather(c)
        d = c - (NBUF - 1)
        if d >= 0:
            db = d % NBUF
            ghandles[db].wait()
            whandles[db] = start_write(d)
    for b in range(NBUF):
        if whandles[b] is not None:
            whandles[b].wait()


@jax.jit
def _lookup(uuid_values_flat, uuid_embedding):
    mesh = plsc.VectorSubcoreMesh(core_axis_name="c", subcore_axis_name="s")
    k = functools.partial(
        pl.kernel,
        mesh=mesh,
        out_type=jax.ShapeDtypeStruct((BS, WIDTH), jnp.float32),
        scratch_types=[
            pltpu.VMEM_SHARED((VOCAB, WIDTH), jnp.float32),
            pltpu.VMEM((B_PER_W,), jnp.int32),
            [pltpu.VMEM((CH, WIDTH), jnp.float32) for _ in range(NBUF)],
            [pltpu.SemaphoreType.DMA for _ in range(NBUF)],
            [pltpu.SemaphoreType.DMA for _ in range(NBUF)],
        ],
        compiler_params=pltpu.CompilerParams(use_tc_tiling_on_sc=False),
    )(_gather_kernel)
    parts = [
        k(uuid_embedding, lax.slice(uuid_values_flat, (s * BS,), ((s + 1) * BS,)))
        for s in range(NSPLIT)
    ]
    return jnp.concatenate(parts, axis=0)


def kernel(partname_indices, pos_values, uuid_values, uuid_embedding):
    flat = _lookup(uuid_values.reshape(-1), uuid_embedding)
    return flat.reshape(N, K * WIDTH)
